# baseline (speedup 1.0000x reference)
import functools

import jax
import jax.numpy as jnp
from jax import lax
from jax.experimental import pallas as pl
from jax.experimental.pallas import tpu as pltpu
from jax.experimental.pallas import tpu_sc as plsc

N = 10000
E = 320000
H = 128
L = 4

NC = 2
NS = 16
NWORK = NC * NS

W = 128
PER_TILE = E // NWORK
IC = 16
NWIN = 80
P = NWIN * W
TRASH = N
NACC = 10112
RPT = NACC // NS


def _seg_body(u_hbm, srcw_hbm, dstw_hbm, zeros_hbm, out_hbm,
              src_c, dst_c, buf_v, acc_sh, gsem):
    c = lax.axis_index("c")
    s = lax.axis_index("s")

    def load_chunk(k):
        cs = (k >> 4) & 1
        ka = pl.multiple_of(k, IC)
        pltpu.sync_copy(srcw_hbm.at[c].at[s].at[pl.ds(ka, IC)], src_c.at[cs])
        pltpu.sync_copy(dstw_hbm.at[c].at[s].at[pl.ds(ka, IC)], dst_c.at[cs])

    def src_win(j):
        return src_c.at[(j >> 4) & 1].at[j & (IC - 1)]

    def dst_win(j):
        return dst_c.at[(j >> 4) & 1].at[j & (IC - 1)]

    def gather(j):
        return pltpu.async_copy(u_hbm.at[src_win(j)], buf_v.at[j & 1], gsem)

    load_chunk(0)
    pltpu.sync_copy(zeros_hbm.at[pl.ds(s * RPT, RPT)],
                    acc_sh.at[pl.ds(s * RPT, RPT)])
    plsc.subcore_barrier()

    def step(j, carry):
        k = j + 1

        @pl.when((k < NWIN) & ((k & (IC - 1)) == 0))
        def _():
            load_chunk(k)

        pltpu.sync_copy(u_hbm.at[src_win(j)], buf_v.at[0])
        pltpu.sync_copy(buf_v.at[0], acc_sh.at[dst_win(j)], add=True)
        return carry

    lax.fori_loop(0, NWIN, step, 0)
    plsc.subcore_barrier()
    pltpu.sync_copy(acc_sh.at[pl.ds(s * RPT, RPT)],
                    out_hbm.at[c].at[pl.ds(s * RPT, RPT)])


@functools.cache
def _get_seg_kernel():
    return pl.kernel(
        _seg_body,
        out_type=jax.ShapeDtypeStruct((NC, NACC, H), jnp.float32),
        mesh=plsc.VectorSubcoreMesh(core_axis_name="c", subcore_axis_name="s",
                                    num_cores=NC, num_subcores=NS),
        scratch_types=[
            pltpu.VMEM((2, IC, W), jnp.int32),
            pltpu.VMEM((2, IC, W), jnp.int32),
            pltpu.VMEM((2, W, H), jnp.float32),
            pltpu.VMEM_SHARED((NACC, H), jnp.float32),
            pltpu.SemaphoreType.DMA,
        ],
    )


def _seg_kernel(u, srcw, dstw, zeros):
    return _get_seg_kernel()(u, srcw, dstw, zeros)


def _cnt_body(dstw_hbm, ones_hbm, zeros_hbm, out_hbm, dst_c, ones_v, acc_sh):
    c = lax.axis_index("c")
    s = lax.axis_index("s")

    def load_chunk(k):
        cs = (k >> 4) & 1
        ka = pl.multiple_of(k, IC)
        pltpu.sync_copy(dstw_hbm.at[c].at[s].at[pl.ds(ka, IC)], dst_c.at[cs])

    def dst_win(j):
        return dst_c.at[(j >> 4) & 1].at[j & (IC - 1)]

    pltpu.sync_copy(ones_hbm, ones_v)
    load_chunk(0)
    pltpu.sync_copy(zeros_hbm.at[pl.ds(s * RPT, RPT)],
                    acc_sh.at[pl.ds(s * RPT, RPT)])
    plsc.subcore_barrier()

    def step(j, carry):
        k = j + 1

        @pl.when((k < NWIN) & ((k & (IC - 1)) == 0))
        def _():
            load_chunk(k)

        pltpu.sync_copy(ones_v, acc_sh.at[dst_win(j)], add=True)
        return carry

    lax.fori_loop(0, NWIN, step, 0)
    plsc.subcore_barrier()
    pltpu.sync_copy(acc_sh.at[pl.ds(s * RPT, RPT)],
                    out_hbm.at[c].at[pl.ds(s * RPT, RPT)])


@functools.cache
def _get_cnt_kernel():
    return pl.kernel(
        _cnt_body,
        out_type=jax.ShapeDtypeStruct((NC, NACC, H), jnp.float32),
        mesh=plsc.VectorSubcoreMesh(core_axis_name="c", subcore_axis_name="s",
                                    num_cores=NC, num_subcores=NS),
        scratch_types=[
            pltpu.VMEM((2, IC, W), jnp.int32),
            pltpu.VMEM((W, H), jnp.float32),
            pltpu.VMEM_SHARED((NACC, H), jnp.float32),
        ],
    )


def _cnt_kernel(dstw, ones, zeros):
    return _get_cnt_kernel()(dstw, ones, zeros)


BM = 1000
HP = lax.Precision.DEFAULT


def _conv_body(do_relu, s_ref, cnt_ref, h_ref, wl_ref, wr_ref, bl_ref, o_ref):
    cntm = jnp.maximum(cnt_ref[0, :, 0:1] + cnt_ref[1, :, 0:1], 1.0)
    mean = (s_ref[0] + s_ref[1]) / cntm
    z = (jnp.dot(mean, wl_ref[...], preferred_element_type=jnp.float32,
                 precision=HP)
         + bl_ref[...]
         + jnp.dot(h_ref[...], wr_ref[...], preferred_element_type=jnp.float32,
                   precision=HP))
    o_ref[...] = jnp.maximum(z, 0.0) if do_relu else z


def _conv_tc(s, cnt, h, wlt, wrt, bl, do_relu):
    return pl.pallas_call(
        functools.partial(_conv_body, do_relu),
        grid=(N // BM,),
        in_specs=[
            pl.BlockSpec((2, BM, H), lambda i: (0, i, 0)),
            pl.BlockSpec((2, BM, H), lambda i: (0, i, 0)),
            pl.BlockSpec((BM, H), lambda i: (i, 0)),
            pl.BlockSpec((H, H), lambda i: (0, 0)),
            pl.BlockSpec((H, H), lambda i: (0, 0)),
            pl.BlockSpec((1, H), lambda i: (0, 0)),
        ],
        out_specs=pl.BlockSpec((BM, H), lambda i: (i, 0)),
        out_shape=jax.ShapeDtypeStruct((N, H), jnp.float32),
    )(s, cnt, h, wlt, wrt, bl)


def _bnrelu_body(z_ref, bn_ref, o_ref):
    z = z_ref[...]
    mu = jnp.mean(z, axis=0, keepdims=True)
    var = jnp.mean((z - mu) * (z - mu), axis=0, keepdims=True)
    zn = (z - mu) / jnp.sqrt(var + 1e-5) * bn_ref[0:1, :] + bn_ref[1:2, :]
    o_ref[...] = jnp.maximum(zn, 0.0)


def _bnrelu(z, bn):
    return pl.pallas_call(
        _bnrelu_body,
        out_shape=jax.ShapeDtypeStruct((N, H), jnp.float32),
    )(z, bn)


def _head_body(h_ref, w0_ref, b0_ref, w1_ref, b1_ref, w2_ref, b2_ref,
               out_ref):
    h = h_ref[...]
    xm = jnp.mean(h, axis=0, keepdims=True)
    xmx = jnp.max(h, axis=0, keepdims=True)
    g = jnp.concatenate([xm, xmx], axis=1)
    g = jnp.maximum(
        jnp.dot(g, w0_ref[...], preferred_element_type=jnp.float32,
                precision=HP) + b0_ref[...], 0.0)
    g = jnp.maximum(
        jnp.dot(g, w1_ref[...], preferred_element_type=jnp.float32,
                precision=HP) + b1_ref[...], 0.0)
    out_ref[...] = (jnp.dot(g, w2_ref[...], preferred_element_type=jnp.float32,
                            precision=HP) + b2_ref[...])


def _head(h, cls):
    return pl.pallas_call(
        _head_body,
        out_shape=jax.ShapeDtypeStruct((1, 1), jnp.float32),
    )(h,
      cls[0]["W"].T, cls[0]["b"][None, :],
      cls[1]["W"].T, cls[1]["b"][None, :],
      cls[2]["W"].T, cls[2]["b"][None, :])


def kernel(x, edge_index, params):
    src = edge_index[0].reshape(NC, NS, PER_TILE)
    dst = edge_index[1].reshape(NC, NS, PER_TILE)
    src = jnp.pad(src, ((0, 0), (0, 0), (0, P - PER_TILE)))
    dst = jnp.pad(dst, ((0, 0), (0, 0), (0, P - PER_TILE)),
                  constant_values=TRASH)
    srcw = src.reshape(NC, NS, NWIN, W)
    dstw = dst.reshape(NC, NS, NWIN, W)

    zeros = jnp.zeros((NACC, H), jnp.float32)
    ones = jnp.ones((W, H), jnp.float32)

    cnt = _cnt_kernel(dstw, ones, zeros)
    h = x
    for i in range(L):
        c = params["convs"][i]
        for li, lin in enumerate((c["l1"], c["l2"])):
            s = _seg_kernel(h, srcw, dstw, zeros)
            h = _conv_tc(s, cnt, h, lin["Wl"].T, lin["Wr"].T,
                         lin["bl"][None, :], do_relu=(li == 0))
        bn = params["bns"][i]
        h = _bnrelu(h, jnp.stack([bn["g"], bn["b"]]))
    return _head(h, params["cls"])


# speedup vs baseline: 1.0976x; 1.0976x over previous
import functools

import jax
import jax.numpy as jnp
from jax import lax
from jax.experimental import pallas as pl
from jax.experimental.pallas import tpu as pltpu
from jax.experimental.pallas import tpu_sc as plsc

N = 10000
E = 320000
H = 128
L = 4

NC = 2
NS = 16
NWORK = NC * NS

W = 128
PER_TILE = E // NWORK
IC = 16
NWIN = 80
P = NWIN * W
TRASH = N
NACC = 10112
RPT = NACC // NS


def _seg_body(u_hbm, srcw_hbm, dstw_hbm, zeros_hbm, out_hbm,
              src_c, dst_c, buf_v, acc_sh, gsem):
    c = lax.axis_index("c")
    s = lax.axis_index("s")

    def load_chunk(k):
        cs = (k >> 4) & 1
        ka = pl.multiple_of(k, IC)
        pltpu.sync_copy(srcw_hbm.at[c].at[s].at[pl.ds(ka, IC)], src_c.at[cs])
        pltpu.sync_copy(dstw_hbm.at[c].at[s].at[pl.ds(ka, IC)], dst_c.at[cs])

    def src_win(j):
        return src_c.at[(j >> 4) & 1].at[j & (IC - 1)]

    def dst_win(j):
        return dst_c.at[(j >> 4) & 1].at[j & (IC - 1)]

    def gather(j):
        return pltpu.async_copy(u_hbm.at[src_win(j)], buf_v.at[j & 1], gsem)

    load_chunk(0)
    pltpu.sync_copy(zeros_hbm.at[pl.ds(s * RPT, RPT)],
                    acc_sh.at[pl.ds(s * RPT, RPT)])
    plsc.subcore_barrier()

    gather(0)

    def step(j, carry):
        b = j & 1
        k = j + 1
        pltpu.make_async_copy(u_hbm.at[src_win(j)], buf_v.at[b], gsem).wait()

        @pl.when(k < NWIN)
        def _():
            @pl.when((k & (IC - 1)) == 0)
            def _():
                load_chunk(k)
            gather(k)

        pltpu.sync_copy(buf_v.at[b], acc_sh.at[dst_win(j)], add=True)
        return carry

    lax.fori_loop(0, NWIN, step, 0)
    plsc.subcore_barrier()
    pltpu.sync_copy(acc_sh.at[pl.ds(s * RPT, RPT)],
                    out_hbm.at[c].at[pl.ds(s * RPT, RPT)])


@functools.cache
def _get_seg_kernel():
    return pl.kernel(
        _seg_body,
        out_type=jax.ShapeDtypeStruct((NC, NACC, H), jnp.float32),
        mesh=plsc.VectorSubcoreMesh(core_axis_name="c", subcore_axis_name="s",
                                    num_cores=NC, num_subcores=NS),
        scratch_types=[
            pltpu.VMEM((2, IC, W), jnp.int32),
            pltpu.VMEM((2, IC, W), jnp.int32),
            pltpu.VMEM((2, W, H), jnp.float32),
            pltpu.VMEM_SHARED((NACC, H), jnp.float32),
            pltpu.SemaphoreType.DMA,
        ],
    )


def _seg_kernel(u, srcw, dstw, zeros):
    return _get_seg_kernel()(u, srcw, dstw, zeros)


def _cnt_body(dstw_hbm, ones_hbm, zeros_hbm, out_hbm, dst_c, ones_v, acc_sh):
    c = lax.axis_index("c")
    s = lax.axis_index("s")

    def load_chunk(k):
        cs = (k >> 4) & 1
        ka = pl.multiple_of(k, IC)
        pltpu.sync_copy(dstw_hbm.at[c].at[s].at[pl.ds(ka, IC)], dst_c.at[cs])

    def dst_win(j):
        return dst_c.at[(j >> 4) & 1].at[j & (IC - 1)]

    pltpu.sync_copy(ones_hbm, ones_v)
    load_chunk(0)
    pltpu.sync_copy(zeros_hbm.at[pl.ds(s * RPT, RPT)],
                    acc_sh.at[pl.ds(s * RPT, RPT)])
    plsc.subcore_barrier()

    def step(j, carry):
        k = j + 1

        @pl.when((k < NWIN) & ((k & (IC - 1)) == 0))
        def _():
            load_chunk(k)

        pltpu.sync_copy(ones_v, acc_sh.at[dst_win(j)], add=True)
        return carry

    lax.fori_loop(0, NWIN, step, 0)
    plsc.subcore_barrier()
    pltpu.sync_copy(acc_sh.at[pl.ds(s * RPT, RPT)],
                    out_hbm.at[c].at[pl.ds(s * RPT, RPT)])


@functools.cache
def _get_cnt_kernel():
    return pl.kernel(
        _cnt_body,
        out_type=jax.ShapeDtypeStruct((NC, NACC, H), jnp.float32),
        mesh=plsc.VectorSubcoreMesh(core_axis_name="c", subcore_axis_name="s",
                                    num_cores=NC, num_subcores=NS),
        scratch_types=[
            pltpu.VMEM((2, IC, W), jnp.int32),
            pltpu.VMEM((W, H), jnp.float32),
            pltpu.VMEM_SHARED((NACC, H), jnp.float32),
        ],
    )


def _cnt_kernel(dstw, ones, zeros):
    return _get_cnt_kernel()(dstw, ones, zeros)


BM = 1000
HP = lax.Precision.DEFAULT


def _conv_body(do_relu, s_ref, cnt_ref, h_ref, wl_ref, wr_ref, bl_ref, o_ref):
    cntm = jnp.maximum(cnt_ref[0, :, 0:1] + cnt_ref[1, :, 0:1], 1.0)
    mean = (s_ref[0] + s_ref[1]) / cntm
    z = (jnp.dot(mean, wl_ref[...], preferred_element_type=jnp.float32,
                 precision=HP)
         + bl_ref[...]
         + jnp.dot(h_ref[...], wr_ref[...], preferred_element_type=jnp.float32,
                   precision=HP))
    o_ref[...] = jnp.maximum(z, 0.0) if do_relu else z


def _conv_tc(s, cnt, h, wlt, wrt, bl, do_relu):
    return pl.pallas_call(
        functools.partial(_conv_body, do_relu),
        grid=(N // BM,),
        in_specs=[
            pl.BlockSpec((2, BM, H), lambda i: (0, i, 0)),
            pl.BlockSpec((2, BM, H), lambda i: (0, i, 0)),
            pl.BlockSpec((BM, H), lambda i: (i, 0)),
            pl.BlockSpec((H, H), lambda i: (0, 0)),
            pl.BlockSpec((H, H), lambda i: (0, 0)),
            pl.BlockSpec((1, H), lambda i: (0, 0)),
        ],
        out_specs=pl.BlockSpec((BM, H), lambda i: (i, 0)),
        out_shape=jax.ShapeDtypeStruct((N, H), jnp.float32),
    )(s, cnt, h, wlt, wrt, bl)


def _bnrelu_body(z_ref, bn_ref, o_ref):
    z = z_ref[...]
    mu = jnp.mean(z, axis=0, keepdims=True)
    var = jnp.mean((z - mu) * (z - mu), axis=0, keepdims=True)
    zn = (z - mu) / jnp.sqrt(var + 1e-5) * bn_ref[0:1, :] + bn_ref[1:2, :]
    o_ref[...] = jnp.maximum(zn, 0.0)


def _bnrelu(z, bn):
    return pl.pallas_call(
        _bnrelu_body,
        out_shape=jax.ShapeDtypeStruct((N, H), jnp.float32),
    )(z, bn)


def _head_body(h_ref, w0_ref, b0_ref, w1_ref, b1_ref, w2_ref, b2_ref,
               out_ref):
    h = h_ref[...]
    xm = jnp.mean(h, axis=0, keepdims=True)
    xmx = jnp.max(h, axis=0, keepdims=True)
    g = jnp.concatenate([xm, xmx], axis=1)
    g = jnp.maximum(
        jnp.dot(g, w0_ref[...], preferred_element_type=jnp.float32,
                precision=HP) + b0_ref[...], 0.0)
    g = jnp.maximum(
        jnp.dot(g, w1_ref[...], preferred_element_type=jnp.float32,
                precision=HP) + b1_ref[...], 0.0)
    out_ref[...] = (jnp.dot(g, w2_ref[...], preferred_element_type=jnp.float32,
                            precision=HP) + b2_ref[...])


def _head(h, cls):
    return pl.pallas_call(
        _head_body,
        out_shape=jax.ShapeDtypeStruct((1, 1), jnp.float32),
    )(h,
      cls[0]["W"].T, cls[0]["b"][None, :],
      cls[1]["W"].T, cls[1]["b"][None, :],
      cls[2]["W"].T, cls[2]["b"][None, :])


def kernel(x, edge_index, params):
    src = edge_index[0].reshape(NC, NS, PER_TILE)
    dst = edge_index[1].reshape(NC, NS, PER_TILE)
    src = jnp.pad(src, ((0, 0), (0, 0), (0, P - PER_TILE)))
    dst = jnp.pad(dst, ((0, 0), (0, 0), (0, P - PER_TILE)),
                  constant_values=TRASH)
    srcw = src.reshape(NC, NS, NWIN, W)
    dstw = dst.reshape(NC, NS, NWIN, W)

    zeros = jnp.zeros((NACC, H), jnp.float32)
    ones = jnp.ones((W, H), jnp.float32)

    cnt = _cnt_kernel(dstw, ones, zeros)
    h = x
    for i in range(L):
        c = params["convs"][i]
        for li, lin in enumerate((c["l1"], c["l2"])):
            s = _seg_kernel(h, srcw, dstw, zeros)
            h = _conv_tc(s, cnt, h, lin["Wl"].T, lin["Wr"].T,
                         lin["bl"][None, :], do_relu=(li == 0))
        bn = params["bns"][i]
        h = _bnrelu(h, jnp.stack([bn["g"], bn["b"]]))
    return _head(h, params["cls"])
